# baseline (speedup 1.0000x reference)
import functools

import jax
import jax.numpy as jnp
from jax import lax
from jax.experimental import pallas as pl
from jax.experimental.pallas import tpu as pltpu
from jax.experimental.pallas import tpu_sc as plsc

_FIB = (1, 1, 2, 3, 5, 8, 13, 21)
_PRIMES = (2654435761, 2246822519, 3266489917, 2028178513, 1220703125,
           1610612741, 805306457, 402653189, 3674653429, 2860486313,
           1073676287, 2971215073, 1500450271, 3267000013, 2654435789,
           4049292737, 2246822531, 3266489927, 2028178519, 1220703133)

_VOCAB = 1024
_BYTE_DIM = 128
_NUM_TABLES = 8
_BUCKETS = 65536
_EPT = 16
_HIDDEN = 512
_B, _T = 64, 512
_N = _B * _T
_ROWS = _NUM_TABLES * _N
_GROUP = 128
_NC, _NS = 2, 16
_NW = _NC * _NS
_TPW = _N // _NW
_TCHUNK = 512
_NCHUNK = _TPW // _TCHUNK
_CROWS = _TCHUNK * _NUM_TABLES
_CGROUPS = _CROWS // _GROUP
_TILE = 512


def _prime_i32(t, k):
    p = int(_PRIMES[(t * 3 + k) % len(_PRIMES)]) % (1 << 32)
    if p >= 1 << 31:
        p -= 1 << 32
    return jnp.int32(p)


def _hash_idx_body(tok_ref, out_ref):
    tok = tok_ref[...]
    shifted = {}
    for off in sorted(set(_FIB)):
        z = jnp.zeros((_B, off), jnp.int32)
        shifted[off] = jnp.concatenate([z, tok[:, : _T - off]], axis=1)
    for t in range(_NUM_TABLES):
        h = jnp.zeros((_B, _T), jnp.int32)
        for k, off in enumerate(_FIB):
            h = h ^ (shifted[off] * _prime_i32(t, k))
        out_ref[t] = h & jnp.int32(_BUCKETS - 1)


_hash_idx = pl.pallas_call(
    _hash_idx_body,
    out_shape=jax.ShapeDtypeStruct((_NUM_TABLES, _B, _T), jnp.int32),
)


_TCOLS = 32
_TBUCK = _TCOLS * 128
_QCH = _BUCKETS // (4 * _TBUCK)


def _sc_transpose_body(x1_hbm, out_hbm, slab_v, rows_v, sem):
    wid = lax.axis_index("s") * _NC + lax.axis_index("c")
    t = wid // 4
    q = wid % 4
    iota = lax.iota(jnp.int32, 16)
    wbase = (lax.shift_right_logical(iota, 1 + 2) * jnp.int32(8 * _TCOLS * 128)
             + lax.bitwise_and(iota, 7) * jnp.int32(128))

    def chunk_body(ci, carry):
        bc0 = q * (4 * _TCOLS) + ci * _TCOLS
        for er in range(2):
            pltpu.sync_copy(
                x1_hbm.at[t, pl.ds((er * 4096 + bc0 * 8) * 128,
                                   _TCOLS * 8 * 128)],
                slab_v.at[pl.ds(er * (8 * _TCOLS * 128), _TCOLS * 8 * 128)])

        def row_body(j, carry2):
            wvec = wbase + (lax.shift_right_logical(j, 7) * jnp.int32(1024)
                            + lax.bitwise_and(j, 127))
            v = plsc.load_gather(slab_v, [wvec])
            rows_v[j] = v
            return carry2

        lax.fori_loop(0, _TBUCK, row_body, 0)
        pltpu.sync_copy(
            rows_v, out_hbm.at[t, pl.ds(q * (4 * _TBUCK) + ci * _TBUCK,
                                        _TBUCK)])
        return carry

    lax.fori_loop(0, _QCH, chunk_body, 0)


@functools.cache
def _build_sc_transpose():
    mesh = plsc.VectorSubcoreMesh(
        core_axis_name="c", subcore_axis_name="s",
        num_cores=_NC, num_subcores=_NS)
    return pl.kernel(
        _sc_transpose_body,
        out_type=jax.ShapeDtypeStruct((_NUM_TABLES, _BUCKETS, _EPT),
                                      jnp.float32),
        mesh=mesh,
        scratch_types=[
            pltpu.VMEM((2 * 8 * _TCOLS * 128,), jnp.float32),
            pltpu.VMEM((_TBUCK, _EPT), jnp.float32),
            pltpu.SemaphoreType.DMA,
        ],
        name="sc_table_transpose",
        compiler_params=pltpu.CompilerParams(
            use_tc_tiling_on_sc=False, needs_layout_passes=False),
    )


def _sc_gather_body(tables_hbm, gidx_hbm, out_hbm, idx_raw, rows_v, sem):
    wid = lax.axis_index("s") * _NC + lax.axis_index("c")

    def chunk_body(ci, carry):
        n0 = wid * _TPW + ci * _TCHUNK
        pltpu.sync_copy(gidx_hbm.at[:, pl.ds(n0, _TCHUNK)], idx_raw)
        for half in range(2):
            descs = []
            for t in range(_NUM_TABLES // 2):
                tt = half * (_NUM_TABLES // 2) + t
                for g in range(_TCHUNK // _GROUP):
                    descs.append(pltpu.async_copy(
                        tables_hbm.at[tt].at[
                            idx_raw.at[tt, pl.ds(g * _GROUP, _GROUP)]],
                        rows_v.at[pl.ds(tt * _TCHUNK + g * _GROUP, _GROUP)],
                        sem))
            for d in descs:
                d.wait()
        for t in range(_NUM_TABLES):
            pltpu.sync_copy(
                rows_v.at[pl.ds(t * _TCHUNK, _TCHUNK)],
                out_hbm.at[pl.ds(n0, _TCHUNK), pl.ds(t * _EPT, _EPT)])
        return carry

    lax.fori_loop(0, _NCHUNK, chunk_body, 0)


@functools.cache
def _build_sc_gather():
    mesh = plsc.VectorSubcoreMesh(
        core_axis_name="c", subcore_axis_name="s",
        num_cores=_NC, num_subcores=_NS)
    return pl.kernel(
        _sc_gather_body,
        out_type=jax.ShapeDtypeStruct((_N, _NUM_TABLES * _EPT), jnp.float32),
        name="sc_embedding_gather",
        mesh=mesh,
        scratch_types=[
            pltpu.VMEM((_NUM_TABLES, _TCHUNK), jnp.int32),
            pltpu.VMEM((_CROWS, _EPT), jnp.float32),
            pltpu.SemaphoreType.DMA,
        ],
        compiler_params=pltpu.CompilerParams(use_tc_tiling_on_sc=False),
    )


def _mm_body(tok_ref, bt_ref, xh_ref, w_ref, b_ref, out_ref):
    tok = tok_ref[0]
    iota_v = lax.broadcasted_iota(jnp.int32, (_VOCAB, _TILE), 0)
    oh = (iota_v == tok).astype(jnp.float32)
    be = lax.dot_general(oh, bt_ref[...], (((0,), (0,)), ((), ())),
                         preferred_element_type=jnp.float32)
    x = jnp.concatenate([be, xh_ref[...]], axis=-1)
    out_ref[...] = (
        jnp.dot(x, w_ref[...], preferred_element_type=jnp.float32) + b_ref[...])


_matmul = pl.pallas_call(
    _mm_body,
    grid=(_N // _TILE,),
    in_specs=[
        pl.BlockSpec((1, 1, _TILE), lambda i: (i, 0, 0)),
        pl.BlockSpec((_VOCAB, _BYTE_DIM), lambda i: (0, 0)),
        pl.BlockSpec((_TILE, _NUM_TABLES * _EPT), lambda i: (i, 0)),
        pl.BlockSpec((_BYTE_DIM + _NUM_TABLES * _EPT, _HIDDEN), lambda i: (0, 0)),
        pl.BlockSpec((1, _HIDDEN), lambda i: (0, 0)),
    ],
    out_specs=pl.BlockSpec((_TILE, _HIDDEN), lambda i: (i, 0)),
    out_shape=jax.ShapeDtypeStruct((_N, _HIDDEN), jnp.float32),
)


def kernel(tokens, byte_table, hash_tables, W, b):
    gidx = _hash_idx(tokens)
    gidx2 = gidx.reshape(_NUM_TABLES, _N)
    x1 = (hash_tables.transpose(0, 2, 1)
          .reshape(_NUM_TABLES, 2, 8, 512, 128)
          .transpose(0, 1, 3, 2, 4)
          .reshape(_NUM_TABLES, 8192 * 128))
    tables_l = _build_sc_transpose()(x1)
    xh = _build_sc_gather()(tables_l, gidx2)
    out = _matmul(tokens.reshape(_N // _TILE, 1, _TILE), byte_table, xh,
                  W, b.reshape(1, _HIDDEN))
    return out.reshape(_B, _T, _HIDDEN)


# speedup vs baseline: 1.0029x; 1.0029x over previous
import functools

import jax
import jax.numpy as jnp
from jax import lax
from jax.experimental import pallas as pl
from jax.experimental.pallas import tpu as pltpu
from jax.experimental.pallas import tpu_sc as plsc

_FIB = (1, 1, 2, 3, 5, 8, 13, 21)
_PRIMES = (2654435761, 2246822519, 3266489917, 2028178513, 1220703125,
           1610612741, 805306457, 402653189, 3674653429, 2860486313,
           1073676287, 2971215073, 1500450271, 3267000013, 2654435789,
           4049292737, 2246822531, 3266489927, 2028178519, 1220703133)

_VOCAB = 1024
_BYTE_DIM = 128
_NUM_TABLES = 8
_BUCKETS = 65536
_EPT = 16
_HIDDEN = 512
_B, _T = 64, 512
_N = _B * _T
_ROWS = _NUM_TABLES * _N
_GROUP = 128
_NC, _NS = 2, 16
_NW = _NC * _NS
_TPW = _N // _NW
_TCHUNK = 512
_NCHUNK = _TPW // _TCHUNK
_CROWS = _TCHUNK * _NUM_TABLES
_CGROUPS = _CROWS // _GROUP
_TILE = 512


def _prime_i32(t, k):
    p = int(_PRIMES[(t * 3 + k) % len(_PRIMES)]) % (1 << 32)
    if p >= 1 << 31:
        p -= 1 << 32
    return jnp.int32(p)


def _hash_idx_body(tok_ref, out_ref):
    tok = tok_ref[...]
    shifted = {}
    for off in sorted(set(_FIB)):
        z = jnp.zeros((_B, off), jnp.int32)
        shifted[off] = jnp.concatenate([z, tok[:, : _T - off]], axis=1)
    for t in range(_NUM_TABLES):
        h = jnp.zeros((_B, _T), jnp.int32)
        for k, off in enumerate(_FIB):
            h = h ^ (shifted[off] * _prime_i32(t, k))
        out_ref[t] = h & jnp.int32(_BUCKETS - 1)


_hash_idx = pl.pallas_call(
    _hash_idx_body,
    out_shape=jax.ShapeDtypeStruct((_NUM_TABLES, _B, _T), jnp.int32),
)


_TCOLS = 32
_TBUCK = _TCOLS * 128
_QCH = _BUCKETS // (4 * _TBUCK)


def _sc_transpose_body(x1_hbm, out_hbm, slab_v, rows_v, sem):
    wid = lax.axis_index("s") * _NC + lax.axis_index("c")
    t = wid // 4
    q = wid % 4
    iota = lax.iota(jnp.int32, 16)
    wbase = (lax.shift_right_logical(iota, 1 + 2) * jnp.int32(8 * _TCOLS * 128)
             + lax.bitwise_and(iota, 7) * jnp.int32(128))

    def chunk_body(ci, carry):
        bc0 = q * (4 * _TCOLS) + ci * _TCOLS
        for er in range(2):
            pltpu.sync_copy(
                x1_hbm.at[t, pl.ds((er * 4096 + bc0 * 8) * 128,
                                   _TCOLS * 8 * 128)],
                slab_v.at[pl.ds(er * (8 * _TCOLS * 128), _TCOLS * 8 * 128)])

        def row_body(g, carry2):
            j0 = g * 16
            base = wbase + (lax.shift_right_logical(j0, 7) * jnp.int32(1024)
                            + lax.bitwise_and(j0, 127))
            for k in range(16):
                v = plsc.load_gather(slab_v, [base + jnp.int32(k)])
                rows_v[j0 + k] = v
            return carry2

        lax.fori_loop(0, _TBUCK // 16, row_body, 0)
        pltpu.sync_copy(
            rows_v, out_hbm.at[t, pl.ds(q * (4 * _TBUCK) + ci * _TBUCK,
                                        _TBUCK)])
        return carry

    lax.fori_loop(0, _QCH, chunk_body, 0)


@functools.cache
def _build_sc_transpose():
    mesh = plsc.VectorSubcoreMesh(
        core_axis_name="c", subcore_axis_name="s",
        num_cores=_NC, num_subcores=_NS)
    return pl.kernel(
        _sc_transpose_body,
        out_type=jax.ShapeDtypeStruct((_NUM_TABLES, _BUCKETS, _EPT),
                                      jnp.float32),
        mesh=mesh,
        scratch_types=[
            pltpu.VMEM((2 * 8 * _TCOLS * 128,), jnp.float32),
            pltpu.VMEM((_TBUCK, _EPT), jnp.float32),
            pltpu.SemaphoreType.DMA,
        ],
        name="sc_table_transpose",
        compiler_params=pltpu.CompilerParams(
            use_tc_tiling_on_sc=False, needs_layout_passes=False),
    )


def _sc_gather_body(tables_hbm, gidx_hbm, out_hbm, idx_raw, rows_v, sem):
    wid = lax.axis_index("s") * _NC + lax.axis_index("c")

    def chunk_body(ci, carry):
        n0 = wid * _TPW + ci * _TCHUNK
        pltpu.sync_copy(gidx_hbm.at[:, pl.ds(n0, _TCHUNK)], idx_raw)
        for half in range(2):
            descs = []
            for t in range(_NUM_TABLES // 2):
                tt = half * (_NUM_TABLES // 2) + t
                for g in range(_TCHUNK // _GROUP):
                    descs.append(pltpu.async_copy(
                        tables_hbm.at[tt].at[
                            idx_raw.at[tt, pl.ds(g * _GROUP, _GROUP)]],
                        rows_v.at[pl.ds(tt * _TCHUNK + g * _GROUP, _GROUP)],
                        sem))
            for d in descs:
                d.wait()
        for t in range(_NUM_TABLES):
            pltpu.sync_copy(
                rows_v.at[pl.ds(t * _TCHUNK, _TCHUNK)],
                out_hbm.at[pl.ds(n0, _TCHUNK), pl.ds(t * _EPT, _EPT)])
        return carry

    lax.fori_loop(0, _NCHUNK, chunk_body, 0)


@functools.cache
def _build_sc_gather():
    mesh = plsc.VectorSubcoreMesh(
        core_axis_name="c", subcore_axis_name="s",
        num_cores=_NC, num_subcores=_NS)
    return pl.kernel(
        _sc_gather_body,
        out_type=jax.ShapeDtypeStruct((_N, _NUM_TABLES * _EPT), jnp.float32),
        name="sc_embedding_gather",
        mesh=mesh,
        scratch_types=[
            pltpu.VMEM((_NUM_TABLES, _TCHUNK), jnp.int32),
            pltpu.VMEM((_CROWS, _EPT), jnp.float32),
            pltpu.SemaphoreType.DMA,
        ],
        compiler_params=pltpu.CompilerParams(use_tc_tiling_on_sc=False),
    )


def _mm_body(tok_ref, bt_ref, xh_ref, w_ref, b_ref, out_ref):
    tok = tok_ref[0]
    iota_v = lax.broadcasted_iota(jnp.int32, (_VOCAB, _TILE), 0)
    oh = (iota_v == tok).astype(jnp.float32)
    be = lax.dot_general(oh, bt_ref[...], (((0,), (0,)), ((), ())),
                         preferred_element_type=jnp.float32)
    x = jnp.concatenate([be, xh_ref[...]], axis=-1)
    out_ref[...] = (
        jnp.dot(x, w_ref[...], preferred_element_type=jnp.float32) + b_ref[...])


_matmul = pl.pallas_call(
    _mm_body,
    grid=(_N // _TILE,),
    in_specs=[
        pl.BlockSpec((1, 1, _TILE), lambda i: (i, 0, 0)),
        pl.BlockSpec((_VOCAB, _BYTE_DIM), lambda i: (0, 0)),
        pl.BlockSpec((_TILE, _NUM_TABLES * _EPT), lambda i: (i, 0)),
        pl.BlockSpec((_BYTE_DIM + _NUM_TABLES * _EPT, _HIDDEN), lambda i: (0, 0)),
        pl.BlockSpec((1, _HIDDEN), lambda i: (0, 0)),
    ],
    out_specs=pl.BlockSpec((_TILE, _HIDDEN), lambda i: (i, 0)),
    out_shape=jax.ShapeDtypeStruct((_N, _HIDDEN), jnp.float32),
)


def kernel(tokens, byte_table, hash_tables, W, b):
    gidx = _hash_idx(tokens)
    gidx2 = gidx.reshape(_NUM_TABLES, _N)
    x1 = (hash_tables.transpose(0, 2, 1)
          .reshape(_NUM_TABLES, 2, 8, 512, 128)
          .transpose(0, 1, 3, 2, 4)
          .reshape(_NUM_TABLES, 8192 * 128))
    tables_l = _build_sc_transpose()(x1)
    xh = _build_sc_gather()(tables_l, gidx2)
    out = _matmul(tokens.reshape(_N // _TILE, 1, _TILE), byte_table, xh,
                  W, b.reshape(1, _HIDDEN))
    return out.reshape(_B, _T, _HIDDEN)


# speedup vs baseline: 1.3044x; 1.3006x over previous
import functools

import jax
import jax.numpy as jnp
from jax import lax
from jax.experimental import pallas as pl
from jax.experimental.pallas import tpu as pltpu
from jax.experimental.pallas import tpu_sc as plsc

_FIB = (1, 1, 2, 3, 5, 8, 13, 21)
_PRIMES = (2654435761, 2246822519, 3266489917, 2028178513, 1220703125,
           1610612741, 805306457, 402653189, 3674653429, 2860486313,
           1073676287, 2971215073, 1500450271, 3267000013, 2654435789,
           4049292737, 2246822531, 3266489927, 2028178519, 1220703133)

_VOCAB = 1024
_BYTE_DIM = 128
_NUM_TABLES = 8
_BUCKETS = 65536
_EPT = 16
_HIDDEN = 512
_B, _T = 64, 512
_N = _B * _T
_ROWS = _NUM_TABLES * _N
_GROUP = 128
_NC, _NS = 2, 16
_NW = _NC * _NS
_TPW = _N // _NW
_TCHUNK = 512
_NCHUNK = _TPW // _TCHUNK
_CROWS = _TCHUNK * _NUM_TABLES
_CGROUPS = _CROWS // _GROUP
_TILE = 512


def _prime_i32(t, k):
    p = int(_PRIMES[(t * 3 + k) % len(_PRIMES)]) % (1 << 32)
    if p >= 1 << 31:
        p -= 1 << 32
    return jnp.int32(p)


def _hash_idx_body(tok_ref, out_ref):
    tok = tok_ref[...]
    shifted = {}
    for off in sorted(set(_FIB)):
        z = jnp.zeros((_B, off), jnp.int32)
        shifted[off] = jnp.concatenate([z, tok[:, : _T - off]], axis=1)
    for t in range(_NUM_TABLES):
        h = jnp.zeros((_B, _T), jnp.int32)
        for k, off in enumerate(_FIB):
            h = h ^ (shifted[off] * _prime_i32(t, k))
        out_ref[t] = h & jnp.int32(_BUCKETS - 1)


_hash_idx = pl.pallas_call(
    _hash_idx_body,
    out_shape=jax.ShapeDtypeStruct((_NUM_TABLES, _B, _T), jnp.int32),
)


_TCOLS = 32
_TBUCK = _TCOLS * 128
_QCH = _BUCKETS // (4 * _TBUCK)


def _sc_transpose_body(x1_hbm, out_hbm, slab_v, rows_v, sem):
    wid = lax.axis_index("s") * _NC + lax.axis_index("c")
    t = wid // 4
    q = wid % 4
    iota = lax.iota(jnp.int32, 16)
    wbase = (lax.shift_right_logical(iota, 1 + 2) * jnp.int32(8 * _TCOLS * 128)
             + lax.bitwise_and(iota, 7) * jnp.int32(128))

    def chunk_body(ci, carry):
        bc0 = q * (4 * _TCOLS) + ci * _TCOLS
        for er in range(2):
            pltpu.sync_copy(
                x1_hbm.at[t, pl.ds((er * 4096 + bc0 * 8) * 128,
                                   _TCOLS * 8 * 128)],
                slab_v.at[pl.ds(er * (8 * _TCOLS * 128), _TCOLS * 8 * 128)])

        @plsc.parallel_loop(0, _TBUCK // 16, unroll=2)
        def row_body(g):
            j0 = g * 16
            base = wbase + (lax.shift_right_logical(j0, 7) * jnp.int32(1024)
                            + lax.bitwise_and(j0, 127))
            for k in range(16):
                v = plsc.load_gather(slab_v, [base + jnp.int32(k)])
                rows_v[j0 + k] = v
        pltpu.sync_copy(
            rows_v, out_hbm.at[t, pl.ds(q * (4 * _TBUCK) + ci * _TBUCK,
                                        _TBUCK)])
        return carry

    lax.fori_loop(0, _QCH, chunk_body, 0)


@functools.cache
def _build_sc_transpose():
    mesh = plsc.VectorSubcoreMesh(
        core_axis_name="c", subcore_axis_name="s",
        num_cores=_NC, num_subcores=_NS)
    return pl.kernel(
        _sc_transpose_body,
        out_type=jax.ShapeDtypeStruct((_NUM_TABLES, _BUCKETS, _EPT),
                                      jnp.float32),
        mesh=mesh,
        scratch_types=[
            pltpu.VMEM((2 * 8 * _TCOLS * 128,), jnp.float32),
            pltpu.VMEM((_TBUCK, _EPT), jnp.float32),
            pltpu.SemaphoreType.DMA,
        ],
        name="sc_table_transpose",
        compiler_params=pltpu.CompilerParams(
            use_tc_tiling_on_sc=False, needs_layout_passes=False),
    )


def _sc_gather_body(tables_hbm, gidx_hbm, out_hbm, idx_raw, rows_v, sem):
    wid = lax.axis_index("s") * _NC + lax.axis_index("c")

    def chunk_body(ci, carry):
        n0 = wid * _TPW + ci * _TCHUNK
        pltpu.sync_copy(gidx_hbm.at[:, pl.ds(n0, _TCHUNK)], idx_raw)
        for half in range(2):
            descs = []
            for t in range(_NUM_TABLES // 2):
                tt = half * (_NUM_TABLES // 2) + t
                for g in range(_TCHUNK // _GROUP):
                    descs.append(pltpu.async_copy(
                        tables_hbm.at[tt].at[
                            idx_raw.at[tt, pl.ds(g * _GROUP, _GROUP)]],
                        rows_v.at[pl.ds(tt * _TCHUNK + g * _GROUP, _GROUP)],
                        sem))
            for d in descs:
                d.wait()
        for t in range(_NUM_TABLES):
            pltpu.sync_copy(
                rows_v.at[pl.ds(t * _TCHUNK, _TCHUNK)],
                out_hbm.at[pl.ds(n0, _TCHUNK), pl.ds(t * _EPT, _EPT)])
        return carry

    lax.fori_loop(0, _NCHUNK, chunk_body, 0)


@functools.cache
def _build_sc_gather():
    mesh = plsc.VectorSubcoreMesh(
        core_axis_name="c", subcore_axis_name="s",
        num_cores=_NC, num_subcores=_NS)
    return pl.kernel(
        _sc_gather_body,
        out_type=jax.ShapeDtypeStruct((_N, _NUM_TABLES * _EPT), jnp.float32),
        name="sc_embedding_gather",
        mesh=mesh,
        scratch_types=[
            pltpu.VMEM((_NUM_TABLES, _TCHUNK), jnp.int32),
            pltpu.VMEM((_CROWS, _EPT), jnp.float32),
            pltpu.SemaphoreType.DMA,
        ],
        compiler_params=pltpu.CompilerParams(use_tc_tiling_on_sc=False),
    )


def _mm_body(tok_ref, bt_ref, xh_ref, w_ref, b_ref, out_ref):
    tok = tok_ref[0]
    iota_v = lax.broadcasted_iota(jnp.int32, (_VOCAB, _TILE), 0)
    oh = (iota_v == tok).astype(jnp.float32)
    be = lax.dot_general(oh, bt_ref[...], (((0,), (0,)), ((), ())),
                         preferred_element_type=jnp.float32)
    x = jnp.concatenate([be, xh_ref[...]], axis=-1)
    out_ref[...] = (
        jnp.dot(x, w_ref[...], preferred_element_type=jnp.float32) + b_ref[...])


_matmul = pl.pallas_call(
    _mm_body,
    grid=(_N // _TILE,),
    in_specs=[
        pl.BlockSpec((1, 1, _TILE), lambda i: (i, 0, 0)),
        pl.BlockSpec((_VOCAB, _BYTE_DIM), lambda i: (0, 0)),
        pl.BlockSpec((_TILE, _NUM_TABLES * _EPT), lambda i: (i, 0)),
        pl.BlockSpec((_BYTE_DIM + _NUM_TABLES * _EPT, _HIDDEN), lambda i: (0, 0)),
        pl.BlockSpec((1, _HIDDEN), lambda i: (0, 0)),
    ],
    out_specs=pl.BlockSpec((_TILE, _HIDDEN), lambda i: (i, 0)),
    out_shape=jax.ShapeDtypeStruct((_N, _HIDDEN), jnp.float32),
)


def kernel(tokens, byte_table, hash_tables, W, b):
    gidx = _hash_idx(tokens)
    gidx2 = gidx.reshape(_NUM_TABLES, _N)
    x1 = (hash_tables.transpose(0, 2, 1)
          .reshape(_NUM_TABLES, 2, 8, 512, 128)
          .transpose(0, 1, 3, 2, 4)
          .reshape(_NUM_TABLES, 8192 * 128))
    tables_l = _build_sc_transpose()(x1)
    xh = _build_sc_gather()(tables_l, gidx2)
    out = _matmul(tokens.reshape(_N // _TILE, 1, _TILE), byte_table, xh,
                  W, b.reshape(1, _HIDDEN))
    return out.reshape(_B, _T, _HIDDEN)
